# Initial kernel scaffold; baseline (speedup 1.0000x reference)
import jax
import jax.numpy as jnp
from jax.experimental import pallas as pl


def kernel(enc1, enc2, emb, W1, b1, W2, b2):
    raise NotImplementedError("write your pallas kernel here")


# speedup vs baseline: 4.3096x; 4.3096x over previous
import functools

import jax
import jax.numpy as jnp
from jax import lax
from jax.experimental import pallas as pl
from jax.experimental.pallas import tpu as pltpu
from jax.experimental.pallas import tpu_sc as plsc

_NC = 2
_NS = 16
_NW = _NC * _NS
_SUB = 128


def _make_sc_pooled_gather(B, L, D):
    b_per_w = B // _NW
    nsub = b_per_w // _SUB
    mesh = plsc.VectorSubcoreMesh(
        core_axis_name="c", subcore_axis_name="s",
        num_cores=_NC, num_subcores=_NS)

    @functools.partial(
        pl.kernel,
        out_type=(jax.ShapeDtypeStruct((B, D), jnp.float32),
                  jax.ShapeDtypeStruct((B, D), jnp.float32)),
        mesh=mesh,
        scratch_types=[
            pltpu.VMEM((L, nsub, _SUB), jnp.int32),
            pltpu.VMEM((L, nsub, _SUB), jnp.int32),
            pltpu.VMEM((b_per_w, D), jnp.float32),
            pltpu.VMEM((b_per_w, D), jnp.float32),
            pltpu.SemaphoreType.DMA,
        ],
        compiler_params=pltpu.CompilerParams(use_tc_tiling_on_sc=False),
    )
    def sc_kernel(enc1t, enc2t, emb_hbm, e1_out, e2_out,
                  idx1_v, idx2_v, acc1_v, acc2_v, sem):
        wid = lax.axis_index("s") * _NC + lax.axis_index("c")
        base = wid * b_per_w

        pltpu.sync_copy(enc1t.at[:, pl.ds(wid * nsub, nsub), :], idx1_v)
        pltpu.sync_copy(enc2t.at[:, pl.ds(wid * nsub, nsub), :], idx2_v)

        def fire(l, add):
            ds = []
            for idx_v, acc_v in ((idx1_v, acc1_v), (idx2_v, acc2_v)):
                for c in range(nsub):
                    ds.append(pltpu.async_copy(
                        emb_hbm.at[idx_v.at[l, c]],
                        acc_v.at[pl.ds(c * _SUB, _SUB)],
                        sem, add=add))
            return ds

        for d in fire(0, False):
            d.wait()

        def body(l, carry):
            for d in fire(l, True):
                d.wait()
            return carry

        lax.fori_loop(1, L, body, 0)

        pltpu.sync_copy(acc1_v, e1_out.at[pl.ds(base, b_per_w)])
        pltpu.sync_copy(acc2_v, e2_out.at[pl.ds(base, b_per_w)])

    return sc_kernel


def _make_tc_mlp(B, L, D, n_types, nhid, nclasses):
    blk = 512
    grid = (B // blk,)

    def body(e1_ref, e2_ref, c1_ref, c2_ref, w1_ref, b1_ref, w2_ref, b2_ref,
             out_ref):
        n1 = jnp.sum((c1_ref[...] != n_types).astype(jnp.float32), axis=1,
                     keepdims=True)
        n2 = jnp.sum((c2_ref[...] != n_types).astype(jnp.float32), axis=1,
                     keepdims=True)
        e1 = e1_ref[...] / n1
        e2 = e2_ref[...] / n2
        feat = jnp.concatenate([e1, e2, e1 * e2, jnp.abs(e1 - e2)], axis=1)
        h = lax.dot_general(feat, w1_ref[...], (((1,), (1,)), ((), ())),
                            preferred_element_type=jnp.float32) + b1_ref[...]
        h = jax.nn.sigmoid(h)
        out_ref[...] = lax.dot_general(h, w2_ref[...], (((1,), (1,)), ((), ())),
                                       preferred_element_type=jnp.float32
                                       ) + b2_ref[...]

    return pl.pallas_call(
        body,
        grid=grid,
        in_specs=[
            pl.BlockSpec((blk, D), lambda i: (i, 0)),
            pl.BlockSpec((blk, D), lambda i: (i, 0)),
            pl.BlockSpec((blk, L), lambda i: (i, 0)),
            pl.BlockSpec((blk, L), lambda i: (i, 0)),
            pl.BlockSpec((nhid, 4 * D), lambda i: (0, 0)),
            pl.BlockSpec((1, nhid), lambda i: (0, 0)),
            pl.BlockSpec((nclasses, nhid), lambda i: (0, 0)),
            pl.BlockSpec((1, nclasses), lambda i: (0, 0)),
        ],
        out_specs=pl.BlockSpec((blk, nclasses), lambda i: (i, 0)),
        out_shape=jax.ShapeDtypeStruct((B, nclasses), jnp.float32),
    )


def kernel(enc1, enc2, emb, W1, b1, W2, b2):
    B, L = enc1.shape
    D = emb.shape[1]
    n_types = emb.shape[0] - 1
    nhid = W1.shape[0]
    nclasses = W2.shape[0]

    enc1t = enc1.T.reshape(L, B // _SUB, _SUB)
    enc2t = enc2.T.reshape(L, B // _SUB, _SUB)

    e1_sum, e2_sum = _make_sc_pooled_gather(B, L, D)(enc1t, enc2t, emb)
    return _make_tc_mlp(B, L, D, n_types, nhid, nclasses)(
        e1_sum, e2_sum, enc1, enc2, W1, b1.reshape(1, nhid),
        W2, b2.reshape(1, nclasses))
